# baseline (speedup 1.0000x reference)
import functools

import jax
import jax.numpy as jnp
from jax import lax
from jax.experimental import pallas as pl
from jax.experimental.pallas import tpu as pltpu
from jax.experimental.pallas import tpu_sc as plsc

VOCAB = 1000000
D = 64
H = 32
B = 4096
SEQ = 200

NC = 2
NS = 16
NW = NC * NS
RW = B // NW
CH = 100
NCHUNK = SEQ // CH
LANES = 16

TCOL = 16384
SH = TCOL.bit_length() - 1
NBLK = (VOCAB + 2 * TCOL - 1) // (2 * TCOL)


RUNROLL = 4


NTOK = RW * SEQ
CHA = 104
CHB = SEQ - CHA


def _sc_pool_body(
    tok_hbm, table_hbm, out_hbm, tok_v, idx_v, bufa_v, bufb_v, pool_v, sem0, sem1
):
    wid = lax.axis_index("s") * NC + lax.axis_index("c")
    pltpu.sync_copy(tok_hbm.at[pl.ds(wid * NTOK, NTOK)], tok_v)

    def xform(i, carry):
        t = tok_v[pl.ds(i * LANES, LANES)]
        idx_v[pl.ds(i * LANES, LANES)] = (
            ((t >> (SH + 1)) << (SH + 1))
            | ((t & (TCOL - 1)) << 1)
            | ((t >> SH) & 1)
        )
        return carry

    lax.fori_loop(0, NTOK // LANES, xform, 0)

    zero = jnp.zeros((LANES,), jnp.float32)
    sems = (sem0, sem1)

    def copies(b, slot):
        return (
            pltpu.make_async_copy(
                table_hbm.at[idx_v.at[pl.ds(b * SEQ, CHA)]],
                bufa_v.at[slot],
                sems[slot],
            ),
            pltpu.make_async_copy(
                table_hbm.at[idx_v.at[pl.ds(b * SEQ + CHA, CHB)]],
                bufb_v.at[slot],
                sems[slot],
            ),
        )

    def issue(b, slot):
        for cp in copies(b, slot):
            cp.start()

    def wait(b, slot):
        for cp in copies(b, slot):
            cp.wait()

    def reduce_row(b, slot):
        def red_both(i, accs):
            accs = list(accs)
            r = i * RUNROLL
            for u in range(RUNROLL):
                for k in range(D // LANES):
                    accs[k] = accs[k] + bufa_v[slot, r + u, pl.ds(LANES * k, LANES)]
                for k in range(D // LANES):
                    accs[k] = accs[k] + bufb_v[slot, r + u, pl.ds(LANES * k, LANES)]
            return tuple(accs)

        def red_tail(i, accs):
            accs = list(accs)
            r = CHB + i * RUNROLL
            for u in range(RUNROLL):
                for k in range(D // LANES):
                    accs[k] = accs[k] + bufa_v[slot, r + u, pl.ds(LANES * k, LANES)]
            return tuple(accs)

        accs = lax.fori_loop(0, CHB // RUNROLL, red_both, (zero,) * (D // LANES))
        accs = lax.fori_loop(0, (CHA - CHB) // RUNROLL, red_tail, accs)
        for k in range(D // LANES):
            pool_v[b, pl.ds(LANES * k, LANES)] = accs[k]

    issue(0, 0)

    def do_pair(i, carry):
        b0 = 2 * i
        b1 = 2 * i + 1
        issue(b1, 1)
        wait(b0, 0)
        reduce_row(b0, 0)

        @pl.when(i < RW // 2 - 1)
        def _():
            issue(b0 + 2, 0)

        wait(b1, 1)
        reduce_row(b1, 1)
        return carry

    lax.fori_loop(0, RW // 2, do_pair, 0)
    pltpu.sync_copy(pool_v, out_hbm.at[pl.ds(wid * RW, RW)])


@functools.partial(jax.jit, static_argnames=())
def _sc_pool(tok2, table):
    mesh = plsc.VectorSubcoreMesh(
        core_axis_name="c", subcore_axis_name="s", num_cores=NC, num_subcores=NS
    )
    return pl.kernel(
        _sc_pool_body,
        out_type=jax.ShapeDtypeStruct((B, D), jnp.float32),
        mesh=mesh,
        scratch_types=[
            pltpu.VMEM((NTOK,), jnp.int32),
            pltpu.VMEM((NTOK,), jnp.int32),
            pltpu.VMEM((2, CHA, D), jnp.float32),
            pltpu.VMEM((2, CHB, D), jnp.float32),
            pltpu.VMEM((RW, D), jnp.float32),
            pltpu.SemaphoreType.DMA,
            pltpu.SemaphoreType.DMA,
        ],
        compiler_params=pltpu.CompilerParams(use_tc_tiling_on_sc=False),
    )(tok2, table)


def _tp_body(ta_ref, tb_ref, out_ref):
    eye = jnp.eye(D, dtype=jnp.float32)
    dn = (((0,), (0,)), ((), ()))
    out_ref[:, 0:D] = lax.dot_general(
        ta_ref[...], eye, dn, preferred_element_type=jnp.float32
    )
    out_ref[:, D : 2 * D] = lax.dot_general(
        tb_ref[...], eye, dn, preferred_element_type=jnp.float32
    )


def _tc_transpose(tableT):
    return pl.pallas_call(
        _tp_body,
        grid=(NBLK,),
        in_specs=[
            pl.BlockSpec((D, TCOL), lambda j: (0, 2 * j)),
            pl.BlockSpec(
                (D, TCOL),
                lambda j: (0, jnp.minimum(2 * j + 1, (VOCAB - 1) // TCOL)),
            ),
        ],
        out_specs=pl.BlockSpec((TCOL, 2 * D), lambda j: (j, 0)),
        out_shape=jax.ShapeDtypeStruct((NBLK * TCOL, 2 * D), jnp.float32),
        compiler_params=pltpu.CompilerParams(
            vmem_limit_bytes=100 * 1024 * 1024
        ),
    )(tableT, tableT)


def _mlp_body(pool_ref, w1_ref, b1_ref, w2_ref, b2_ref, out_ref):
    pooled = pool_ref[...] * (1.0 / SEQ)
    h = jnp.dot(pooled, w1_ref[...], preferred_element_type=jnp.float32)
    h = jnp.maximum(h + b1_ref[...], 0.0)
    out_ref[...] = jnp.sum(h * w2_ref[...], axis=1, keepdims=True) + b2_ref[...]


@jax.jit
def _mlp(pooled, W1, b1r, W2r, b2r):
    return pl.pallas_call(
        _mlp_body,
        out_shape=jax.ShapeDtypeStruct((B, 1), jnp.float32),
    )(pooled, W1, b1r, W2r, b2r)


def kernel(tokens, table, W1, b1, W2, b2):
    tok_flat = tokens.reshape(B * SEQ).astype(jnp.int32)
    packed = _tc_transpose(table.T)
    pooled = _sc_pool(tok_flat, packed.reshape(2 * NBLK * TCOL, D))
    out = _mlp(
        pooled,
        W1,
        b1.reshape(1, H),
        W2.reshape(1, H),
        b2.reshape(1, 1),
    )
    return out[:, 0]


# speedup vs baseline: 1.0063x; 1.0063x over previous
import functools

import jax
import jax.numpy as jnp
from jax import lax
from jax.experimental import pallas as pl
from jax.experimental.pallas import tpu as pltpu
from jax.experimental.pallas import tpu_sc as plsc

VOCAB = 1000000
D = 64
H = 32
B = 4096
SEQ = 200

NC = 2
NS = 16
NW = NC * NS
RW = B // NW
CH = 100
NCHUNK = SEQ // CH
LANES = 16

TCOL = 16384
SH = TCOL.bit_length() - 1
NBLK = (VOCAB + 2 * TCOL - 1) // (2 * TCOL)


RUNROLL = 4


NTOK = RW * SEQ
CHA = 104
CHB = SEQ - CHA


def _sc_pool_body(
    tok_hbm, table_hbm, out_hbm, tok_v, idx_v, bufa_v, bufb_v, pool_v, sem0, sem1
):
    wid = lax.axis_index("s") * NC + lax.axis_index("c")
    pltpu.sync_copy(tok_hbm.at[pl.ds(wid * NTOK, NTOK)], tok_v)

    def xform(i, carry):
        t = tok_v[pl.ds(i * LANES, LANES)]
        idx_v[pl.ds(i * LANES, LANES)] = (
            ((t >> (SH + 1)) << (SH + 1))
            | ((t & (TCOL - 1)) << 1)
            | ((t >> SH) & 1)
        )
        return carry

    lax.fori_loop(0, NTOK // LANES, xform, 0)

    zero = jnp.zeros((LANES,), jnp.float32)
    sems = (sem0, sem1)

    def copies(b, slot):
        return (
            pltpu.make_async_copy(
                table_hbm.at[idx_v.at[pl.ds(b * SEQ, CHA)]],
                bufa_v.at[slot],
                sems[slot],
            ),
            pltpu.make_async_copy(
                table_hbm.at[idx_v.at[pl.ds(b * SEQ + CHA, CHB)]],
                bufb_v.at[slot],
                sems[slot],
            ),
        )

    def issue(b, slot):
        for cp in copies(b, slot):
            cp.start()

    def wait(b, slot):
        for cp in copies(b, slot):
            cp.wait()

    def reduce_row(b, slot):
        def red_both(i, accs):
            accs = list(accs)
            r = i * RUNROLL
            for u in range(RUNROLL):
                for k in range(D // LANES):
                    accs[k] = accs[k] + bufa_v[slot, r + u, pl.ds(LANES * k, LANES)]
                for k in range(D // LANES):
                    accs[k] = accs[k] + bufb_v[slot, r + u, pl.ds(LANES * k, LANES)]
            return tuple(accs)

        def red_tail(i, accs):
            accs = list(accs)
            r = CHB + i * RUNROLL
            for u in range(RUNROLL):
                for k in range(D // LANES):
                    accs[k] = accs[k] + bufa_v[slot, r + u, pl.ds(LANES * k, LANES)]
            return tuple(accs)

        accs = lax.fori_loop(0, CHB // RUNROLL, red_both, (zero,) * (D // LANES))
        accs = lax.fori_loop(0, (CHA - CHB) // RUNROLL, red_tail, accs)
        for k in range(D // LANES):
            pool_v[b, pl.ds(LANES * k, LANES)] = accs[k]

    issue(0, 0)

    def do_pair(i, carry):
        b0 = 2 * i
        b1 = 2 * i + 1
        issue(b1, 1)
        wait(b0, 0)
        reduce_row(b0, 0)

        @pl.when(i < RW // 2 - 1)
        def _():
            issue(b0 + 2, 0)

        wait(b1, 1)
        reduce_row(b1, 1)
        return carry

    lax.fori_loop(0, RW // 2, do_pair, 0)
    pltpu.sync_copy(pool_v, out_hbm.at[pl.ds(wid * RW, RW)])


@functools.partial(jax.jit, static_argnames=())
def _sc_pool(tok2, table):
    mesh = plsc.VectorSubcoreMesh(
        core_axis_name="c", subcore_axis_name="s", num_cores=NC, num_subcores=NS
    )
    return pl.kernel(
        _sc_pool_body,
        out_type=jax.ShapeDtypeStruct((B, D), jnp.float32),
        mesh=mesh,
        scratch_types=[
            pltpu.VMEM((NTOK,), jnp.int32),
            pltpu.VMEM((NTOK,), jnp.int32),
            pltpu.VMEM((2, CHA, D), jnp.float32),
            pltpu.VMEM((2, CHB, D), jnp.float32),
            pltpu.VMEM((RW, D), jnp.float32),
            pltpu.SemaphoreType.DMA,
            pltpu.SemaphoreType.DMA,
        ],
        compiler_params=pltpu.CompilerParams(use_tc_tiling_on_sc=False),
    )(tok2, table)


def _tp_body(t_ref, out_ref):
    out_ref[:, 0:D] = jnp.swapaxes(t_ref[:, 0:TCOL], 0, 1)
    out_ref[:, D : 2 * D] = jnp.swapaxes(t_ref[:, TCOL : 2 * TCOL], 0, 1)


def _tc_transpose(tableT):
    return pl.pallas_call(
        _tp_body,
        grid=(NBLK,),
        in_specs=[pl.BlockSpec((D, 2 * TCOL), lambda j: (0, j))],
        out_specs=pl.BlockSpec((TCOL, 2 * D), lambda j: (j, 0)),
        out_shape=jax.ShapeDtypeStruct((NBLK * TCOL, 2 * D), jnp.float32),
        compiler_params=pltpu.CompilerParams(
            vmem_limit_bytes=100 * 1024 * 1024
        ),
    )(tableT)


def _mlp_body(pool_ref, w1_ref, b1_ref, w2_ref, b2_ref, out_ref):
    pooled = pool_ref[...] * (1.0 / SEQ)
    h = jnp.dot(pooled, w1_ref[...], preferred_element_type=jnp.float32)
    h = jnp.maximum(h + b1_ref[...], 0.0)
    out_ref[...] = jnp.sum(h * w2_ref[...], axis=1, keepdims=True) + b2_ref[...]


@jax.jit
def _mlp(pooled, W1, b1r, W2r, b2r):
    return pl.pallas_call(
        _mlp_body,
        out_shape=jax.ShapeDtypeStruct((B, 1), jnp.float32),
    )(pooled, W1, b1r, W2r, b2r)


def kernel(tokens, table, W1, b1, W2, b2):
    tok_flat = tokens.reshape(B * SEQ).astype(jnp.int32)
    packed = _tc_transpose(table.T)
    pooled = _sc_pool(tok_flat, packed.reshape(2 * NBLK * TCOL, D))
    out = _mlp(
        pooled,
        W1,
        b1.reshape(1, H),
        W2.reshape(1, H),
        b2.reshape(1, 1),
    )
    return out[:, 0]
